# Initial kernel scaffold; baseline (speedup 1.0000x reference)
import jax
import jax.numpy as jnp
from jax.experimental import pallas as pl


def kernel(X_train, X_test, y_train):
    raise NotImplementedError("write your pallas kernel here")


# speedup vs baseline: 5.9368x; 5.9368x over previous
import functools

import jax
import jax.numpy as jnp
from jax import lax
from jax.experimental import pallas as pl
from jax.experimental.pallas import tpu as pltpu
from jax.experimental.pallas import tpu_sc as plsc

K_NEIGH = 5
N_CLASSES = 100
Q = 1024
D = 32
KB = 2048
NB = 50
KPAD = KB * NB
QT = 128
PAD_VAL = 1.0e4
NW = 32
QPW = Q // NW


def _stage1_body(xt_ref, q_ref, m1_ref, m2_ref, i1_ref, i2_ref):
    b = pl.program_id(0)
    q = q_ref[...]
    xt = xt_ref[...]
    qsq = jnp.sum(q * q, axis=1, keepdims=True)
    ksq = jnp.sum(xt * xt, axis=0, keepdims=True)
    g = jnp.dot(q, xt, preferred_element_type=jnp.float32)
    d2 = (qsq + ksq) - 2.0 * g
    lane = lax.broadcasted_iota(jnp.int32, (Q, 128), 1)
    base = b * KB
    m1 = jnp.full((Q, 128), jnp.inf, jnp.float32)
    m2 = m1
    i1 = jnp.zeros((Q, 128), jnp.int32)
    i2 = i1
    for r in range(KB // 128):
        d = d2[:, r * 128:(r + 1) * 128]
        iv = lane + (base + r * 128)
        c1 = d < m1
        c2 = d < m2
        m2 = jnp.where(c1, m1, jnp.where(c2, d, m2))
        i2 = jnp.where(c1, i1, jnp.where(c2, iv, i2))
        m1 = jnp.where(c1, d, m1)
        i1 = jnp.where(c1, iv, i1)
    m1_ref[...] = m1[None]
    m2_ref[...] = m2[None]
    i1_ref[...] = i1[None]
    i2_ref[...] = i2[None]


def _stage1(xt, xtest):
    return pl.pallas_call(
        _stage1_body,
        grid=(NB,),
        in_specs=[
            pl.BlockSpec((D, KB), lambda b: (0, b)),
            pl.BlockSpec((Q, D), lambda b: (0, 0)),
        ],
        out_specs=[
            pl.BlockSpec((1, Q, 128), lambda b: (b, 0, 0)),
            pl.BlockSpec((1, Q, 128), lambda b: (b, 0, 0)),
            pl.BlockSpec((1, Q, 128), lambda b: (b, 0, 0)),
            pl.BlockSpec((1, Q, 128), lambda b: (b, 0, 0)),
        ],
        out_shape=[
            jax.ShapeDtypeStruct((NB, Q, 128), jnp.float32),
            jax.ShapeDtypeStruct((NB, Q, 128), jnp.float32),
            jax.ShapeDtypeStruct((NB, Q, 128), jnp.int32),
            jax.ShapeDtypeStruct((NB, Q, 128), jnp.int32),
        ],
    )(xt, xtest)


def _stage2_body(m1_ref, m2_ref, i1_ref, i2_ref, out_ref):
    v1 = m1_ref[...]
    v2 = m2_ref[...]
    j1 = i1_ref[...]
    j2 = i2_ref[...]
    big = jnp.int32(2 ** 30)
    col = lax.broadcasted_iota(jnp.int32, (QT, 8), 1)
    acc = jnp.zeros((QT, 8), jnp.int32)
    for j in range(K_NEIGH):
        a = jnp.minimum(jnp.min(v1, axis=0), jnp.min(v2, axis=0))
        bv = jnp.min(a, axis=1, keepdims=True)
        bvb = bv[None]
        b1 = jnp.min(jnp.where(v1 == bvb, j1, big), axis=0)
        b2 = jnp.min(jnp.where(v2 == bvb, j2, big), axis=0)
        bi = jnp.min(jnp.minimum(b1, b2), axis=1, keepdims=True)
        acc = jnp.where(col == j, bi, acc)
        bib = bi[None]
        v1 = jnp.where((v1 == bvb) & (j1 == bib), jnp.inf, v1)
        v2 = jnp.where((v2 == bvb) & (j2 == bib), jnp.inf, v2)
    out_ref[...] = acc


def _stage2(m1, m2, i1, i2):
    spec = pl.BlockSpec((NB, QT, 128), lambda t: (0, t, 0))
    return pl.pallas_call(
        _stage2_body,
        grid=(Q // QT,),
        in_specs=[spec, spec, spec, spec],
        out_specs=pl.BlockSpec((QT, 8), lambda t: (t, 0)),
        out_shape=jax.ShapeDtypeStruct((Q, 8), jnp.int32),
    )(m1, m2, i1, i2)


def _stage3_body(idx_hbm, y_hbm, out_hbm, y_v, idx_v, out_v):
    c = lax.axis_index("c")
    s = lax.axis_index("s")
    w = s * 2 + c
    qb = w * QPW
    pltpu.sync_copy(y_hbm, y_v)
    pltpu.sync_copy(idx_hbm, idx_v)
    for g in range(QPW // 16):
        labels = []
        for j in range(K_NEIGH):
            idx = idx_v[j, pl.ds(qb + g * 16, 16)]
            labels.append(plsc.load_gather(y_v, [idx]))
        counts = []
        for j in range(K_NEIGH):
            cj = jnp.zeros((16,), jnp.int32)
            for k in range(K_NEIGH):
                cj = cj + jnp.where(labels[j] == labels[k], 1, 0)
            counts.append(cj)
        pred = labels[0]
        best = counts[0] * (N_CLASSES * 10) - labels[0]
        for j in range(1, K_NEIGH):
            sc = counts[j] * (N_CLASSES * 10) - labels[j]
            take = sc > best
            pred = jnp.where(take, labels[j], pred)
            best = jnp.where(take, sc, best)
        out_v[pl.ds(g * 16, 16)] = pred
    pltpu.sync_copy(out_v, out_hbm.at[pl.ds(qb, QPW)])


def _stage3(idx_t, y_train):
    mesh = plsc.VectorSubcoreMesh(core_axis_name="c", subcore_axis_name="s")
    n_train = y_train.shape[0]
    fn = functools.partial(
        pl.kernel,
        mesh=mesh,
        out_type=jax.ShapeDtypeStruct((Q,), jnp.int32),
        scratch_types=[
            pltpu.VMEM((n_train,), jnp.int32),
            pltpu.VMEM((8, Q), jnp.int32),
            pltpu.VMEM((QPW,), jnp.int32),
        ],
        compiler_params=pltpu.CompilerParams(needs_layout_passes=False),
    )(_stage3_body)
    return fn(idx_t, y_train)


def kernel(X_train, X_test, y_train):
    n_train = X_train.shape[0]
    pad = KPAD - n_train
    xp = jnp.concatenate(
        [X_train, jnp.full((pad, D), PAD_VAL, jnp.float32)], axis=0)
    xt = xp.T
    m1, m2, i1, i2 = _stage1(xt, X_test)
    top = _stage2(m1, m2, i1, i2)
    idx_t = top.T
    return _stage3(idx_t, y_train)
